# baseline (speedup 1.0000x reference)
import functools

import jax
import jax.numpy as jnp
from jax import lax
from jax.experimental import pallas as pl
from jax.experimental.pallas import tpu as pltpu
from jax.experimental.pallas import tpu_sc as plsc

N = 50000
E = 800000
K = 8

CH = 128
EP = 802816
NA = 50176
BLK = 2000
NB = N // BLK
CNT = 8.0 * N


def _seg_kernel(nch, wpad, split_features, tab_hbm, idxq_hbm, zeros_hbm,
                out_hbm, d0, d1, d2, d3, r0, r1, r2, r3, acc_sh,
                i0, i1, i2, i3, g0, g1, g2, g3, s0, s1, s2, s3):
    sd = [d0, d1, d2, d3]
    rows = [r0, r1, r2, r3]
    isem = [i0, i1, i2, i3]
    gsem = [g0, g1, g2, g3]
    ssem = [s0, s1, s2, s3]
    c = lax.axis_index("c")
    s = lax.axis_index("s")
    zr = NA // 16
    pltpu.sync_copy(zeros_hbm.at[pl.ds(s * zr, zr)], acc_sh.at[pl.ds(s * zr, zr)])
    plsc.subcore_barrier()
    if split_features:
        cb = (c * EP + s * (EP // 16)) // CH
    else:
        cb = ((c * 16 + s) * (EP // 32)) // CH

    def fire_idx(jj, b):
        pltpu.async_copy(idxq_hbm.at[cb + jj], sd[b], isem[b])

    def wait_idx(jj, b):
        pltpu.make_async_copy(idxq_hbm.at[cb + jj], sd[b], isem[b]).wait()

    def fire_gather(jj, b):
        pltpu.async_copy(tab_hbm.at[sd[b].at[0]], rows[b], gsem[b])

    def wait_gather(jj, b):
        pltpu.make_async_copy(tab_hbm.at[sd[b].at[0]], rows[b], gsem[b]).wait()

    def fire_scatter(jj, b):
        pltpu.async_copy(rows[b], acc_sh.at[sd[b].at[1]], ssem[b], add=True)

    def wait_scatter(jj, b):
        pltpu.make_async_copy(rows[b], acc_sh.at[sd[b].at[1]], ssem[b]).wait()

    def body(i, _):
        for b in range(4):
            jj = i * 4 + b

            @pl.when(i >= 1)
            def _ws():
                wait_scatter(jj - 4, b)
            fire_idx(jj, b)
            b2 = (b + 2) % 4
            b3 = (b + 1) % 4
            if b >= 2:
                wait_idx(jj - 2, b2)
                fire_gather(jj - 2, b2)
            else:
                @pl.when(i >= 1)
                def _wg():
                    wait_idx(jj - 2, b2)
                    fire_gather(jj - 2, b2)
            if b >= 3:
                wait_gather(jj - 3, b3)
                fire_scatter(jj - 3, b3)
            else:
                @pl.when(i >= 1)
                def _wsc():
                    wait_gather(jj - 3, b3)
                    fire_scatter(jj - 3, b3)
        return _

    lax.fori_loop(0, nch // 4, body, None)
    for jj in (nch - 2, nch - 1):
        b = jj % 4
        wait_idx(jj, b)
        fire_gather(jj, b)
    for jj in (nch - 3, nch - 2, nch - 1):
        b = jj % 4
        wait_gather(jj, b)
        fire_scatter(jj, b)
    for jj in range(nch - 4, nch):
        wait_scatter(jj, jj % 4)
    plsc.subcore_barrier()
    pltpu.sync_copy(acc_sh.at[pl.ds(s * zr, zr)],
                    out_hbm.at[c].at[pl.ds(s * zr, zr)])


def _make_seg(nch, wpad, split_features):
    mesh = plsc.VectorSubcoreMesh(core_axis_name="c", subcore_axis_name="s")
    dma = pltpu.SemaphoreType.DMA
    return pl.kernel(
        functools.partial(_seg_kernel, nch, wpad, split_features),
        out_type=jax.ShapeDtypeStruct((2, NA, wpad), jnp.float32),
        mesh=mesh,
        scratch_types=(
            [pltpu.VMEM((2, CH), jnp.int32)] * 4
            + [pltpu.VMEM((CH, wpad), jnp.float32)] * 4
            + [pltpu.VMEM_SHARED((NA, wpad), jnp.float32)]
            + [dma] * 12
        ),
        compiler_params=pltpu.CompilerParams(use_tc_tiling_on_sc=False),
    )


_seg64 = _make_seg(EP // (16 * CH), 32, True)
_seg8 = _make_seg(EP // (32 * CH), 16, False)


def _embed_body(x_ref, wh_ref, bh_ref, o_ref):
    o_ref[...] = jnp.dot(x_ref[...], wh_ref[...], preferred_element_type=jnp.float32) + bh_ref[...]


def _edge_body(ea_ref, we_ref, be_ref, o_ref):
    o_ref[...] = ea_ref[...] * we_ref[...] + be_ref[...]


def _embed_h(x, Wh, bh):
    n, d = x.shape
    dout = Wh.shape[1]
    return pl.pallas_call(
        _embed_body,
        grid=(n // BLK,),
        in_specs=[
            pl.BlockSpec((BLK, d), lambda i: (i, 0)),
            pl.BlockSpec((d, dout), lambda i: (0, 0)),
            pl.BlockSpec((1, dout), lambda i: (0, 0)),
        ],
        out_specs=pl.BlockSpec((BLK, dout), lambda i: (i, 0)),
        out_shape=jax.ShapeDtypeStruct((n, dout), jnp.float32),
    )(x, Wh, bh.reshape(1, dout))


def _embed_e(edge_attr, We, be):
    e = edge_attr.shape[0]
    dout = We.shape[1]
    blk = 8000
    return pl.pallas_call(
        _edge_body,
        grid=(e // blk,),
        in_specs=[
            pl.BlockSpec((blk, 1), lambda i: (i, 0)),
            pl.BlockSpec((1, dout), lambda i: (0, 0)),
            pl.BlockSpec((1, dout), lambda i: (0, 0)),
        ],
        out_specs=pl.BlockSpec((blk, dout), lambda i: (i, 0)),
        out_shape=jax.ShapeDtypeStruct((e, dout), jnp.float32),
    )(edge_attr.reshape(e, 1), We.reshape(1, dout), be.reshape(1, dout))


def _fold_stats(s1, s2, bmat):
    mu_w = jnp.dot(s1, bmat, preferred_element_type=jnp.float32,
                   precision=lax.Precision.HIGHEST) / CNT
    ex2_w = jnp.dot(s2, bmat, preferred_element_type=jnp.float32,
                    precision=lax.Precision.HIGHEST) / CNT
    inv_w = lax.rsqrt(ex2_w - mu_w * mu_w + 1e-5)
    return mu_w, inv_w


def _dense1_body(pe_ref, agg_ref, wk1_ref, b1_ref, wk2_ref, b2_ref, bm_ref,
                 op_ref, om_ref, st_ref):
    p = pl.program_id(0)
    i = pl.program_id(1)
    z = pe_ref[...]
    z = jnp.where(jnp.isnan(z), 0.0, z)
    m = z + agg_ref[0, :, :8] + agg_ref[1, :, :8]
    v = jnp.concatenate(
        [m[:, k:k + 1] * wk1_ref[...] + b1_ref[...] for k in range(8)], axis=1)

    @pl.when((p == 0) & (i == 0))
    def _init():
        st_ref[...] = jnp.zeros_like(st_ref)

    @pl.when(p == 0)
    def _stats():
        st_ref[0:1, :] += jnp.sum(v, axis=0, keepdims=True)
        st_ref[1:2, :] += jnp.sum(v * v, axis=0, keepdims=True)

    @pl.when(p == 1)
    def _out():
        mu_w, inv_w = _fold_stats(st_ref[0:1, :], st_ref[1:2, :], bm_ref[...])
        u = (v - mu_w) * inv_w
        def mlp2(uu):
            return jnp.concatenate(
                [jnp.dot(uu[:, g * 8:(g + 1) * 8], wk2_ref[...],
                         preferred_element_type=jnp.float32) + b2_ref[...]
                 for g in range(8)], axis=1)
        outp = mlp2(jnp.maximum(u, 0.0))
        outm = mlp2(jnp.maximum(-u, 0.0))
        op_ref[0] = outp[:, :32]
        op_ref[1] = outp[:, 32:]
        om_ref[0] = outm[:, :32]
        om_ref[1] = outm[:, 32:]


def _dense1(pe, agg1, wk1, b1t, wk2, b2t, bm):
    spec_tab = pl.BlockSpec((2, BLK, 32), lambda p, i: (0, i, 0))
    return pl.pallas_call(
        _dense1_body,
        grid=(2, NB),
        in_specs=[
            pl.BlockSpec((BLK, 8), lambda p, i: (i, 0)),
            pl.BlockSpec((2, BLK, 16), lambda p, i: (0, i, 0)),
            pl.BlockSpec((1, 8), lambda p, i: (0, 0)),
            pl.BlockSpec((1, 8), lambda p, i: (0, 0)),
            pl.BlockSpec((8, 8), lambda p, i: (0, 0)),
            pl.BlockSpec((1, 8), lambda p, i: (0, 0)),
            pl.BlockSpec((64, 64), lambda p, i: (0, 0)),
        ],
        out_specs=[spec_tab, spec_tab],
        out_shape=[jax.ShapeDtypeStruct((2, N, 32), jnp.float32)] * 2,
        scratch_shapes=[pltpu.VMEM((2, 64), jnp.float32)],
    )(pe, agg1, wk1, b1t, wk2, b2t, bm)


def _dense23_body(outw, tp_ref, aggp_ref, tm_ref, aggm_ref,
                  wk1_ref, b1_ref, wk2_ref, b2_ref, bm_ref,
                  op_ref, om_ref, st_ref):
    p = pl.program_id(0)
    i = pl.program_id(1)

    def halves(t_ref, agg_ref):
        out = []
        for c in (0, 1):
            m = t_ref[c] + agg_ref[c]
            out.append(jnp.concatenate(
                [jnp.dot(m[:, g * 8:(g + 1) * 8], wk1_ref[...],
                         preferred_element_type=jnp.float32) + b1_ref[...]
                 for g in range(4)], axis=1))
        return out

    vp = halves(tp_ref, aggp_ref)
    vm = halves(tm_ref, aggm_ref)

    @pl.when((p == 0) & (i == 0))
    def _init():
        st_ref[...] = jnp.zeros_like(st_ref)

    @pl.when(p == 0)
    def _stats():
        st_ref[0:1, :] += jnp.sum(vp[0] + vp[1], axis=0, keepdims=True)
        st_ref[1:2, :] += jnp.sum(vp[0] * vp[0] + vp[1] * vp[1], axis=0, keepdims=True)
        st_ref[2:3, :] += jnp.sum(vm[0] + vm[1], axis=0, keepdims=True)
        st_ref[3:4, :] += jnp.sum(vm[0] * vm[0] + vm[1] * vm[1], axis=0, keepdims=True)

    @pl.when(p == 1)
    def _out():
        for v, o_ref, r0, r1 in ((vp, op_ref, 0, 1), (vm, om_ref, 2, 3)):
            mu_w, inv_w = _fold_stats(st_ref[r0:r0 + 1, :], st_ref[r1:r1 + 1, :],
                                      bm_ref[...])
            for c in (0, 1):
                u = jnp.maximum((v[c] - mu_w) * inv_w, 0.0)
                o_ref[c] = jnp.concatenate(
                    [jnp.dot(u[:, g * 8:(g + 1) * 8], wk2_ref[...],
                             preferred_element_type=jnp.float32) + b2_ref[...]
                     for g in range(4)], axis=1)


def _dense23(outw, tp, aggp, tm, aggm, wk1, b1t, wk2, b2t, bm):
    spec_in = pl.BlockSpec((2, BLK, 32), lambda p, i: (0, i, 0))
    spec_out = pl.BlockSpec((2, BLK, outw), lambda p, i: (0, i, 0))
    return pl.pallas_call(
        functools.partial(_dense23_body, outw),
        grid=(2, NB),
        in_specs=[
            spec_in, spec_in, spec_in, spec_in,
            pl.BlockSpec((8, 8), lambda p, i: (0, 0)),
            pl.BlockSpec((1, 8), lambda p, i: (0, 0)),
            pl.BlockSpec((8, outw // 4), lambda p, i: (0, 0)),
            pl.BlockSpec((1, outw // 4), lambda p, i: (0, 0)),
            pl.BlockSpec((32, 32), lambda p, i: (0, 0)),
        ],
        out_specs=[spec_out, spec_out],
        out_shape=[jax.ShapeDtypeStruct((2, N, outw), jnp.float32)] * 2,
        scratch_shapes=[pltpu.VMEM((4, 32), jnp.float32)],
    )(tp, aggp, tm, aggm, wk1, b1t, wk2, b2t, bm)


def _rho_body(tp_ref, tm_ref, rw1_ref, rb1_ref, rw2_ref, rb2_ref, o_ref, st_ref):
    p = pl.program_id(0)
    i = pl.program_id(1)
    h = jnp.concatenate([tp_ref[0] + tm_ref[0], tp_ref[1] + tm_ref[1]], axis=1)
    r = jnp.dot(h, rw1_ref[...], preferred_element_type=jnp.float32) + rb1_ref[...]

    @pl.when((p == 0) & (i == 0))
    def _init():
        st_ref[...] = jnp.zeros_like(st_ref)

    @pl.when(p == 0)
    def _stats():
        st_ref[0:1, :] += jnp.sum(r, axis=0, keepdims=True)
        st_ref[1:2, :] += jnp.sum(r * r, axis=0, keepdims=True)

    @pl.when(p == 1)
    def _out():
        mu = st_ref[0:1, :] / N
        ex2 = st_ref[1:2, :] / N
        inv = lax.rsqrt(ex2 - mu * mu + 1e-5)
        u = jnp.maximum((r - mu) * inv, 0.0)
        o_ref[...] = jnp.dot(u, rw2_ref[...], preferred_element_type=jnp.float32) + rb2_ref[...]


def _rho(tp, tm, rw1, rb1, rw2, rb2):
    spec_in = pl.BlockSpec((2, BLK, 16), lambda p, i: (0, i, 0))
    return pl.pallas_call(
        _rho_body,
        grid=(2, NB),
        in_specs=[
            spec_in, spec_in,
            pl.BlockSpec((32, 8), lambda p, i: (0, 0)),
            pl.BlockSpec((1, 8), lambda p, i: (0, 0)),
            pl.BlockSpec((8, 16), lambda p, i: (0, 0)),
            pl.BlockSpec((1, 16), lambda p, i: (0, 0)),
        ],
        out_specs=pl.BlockSpec((BLK, 16), lambda p, i: (i, 0)),
        out_shape=jax.ShapeDtypeStruct((N, 16), jnp.float32),
        scratch_shapes=[pltpu.VMEM((2, 8), jnp.float32)],
    )(tp, tm, rw1, rb1.reshape(1, 8), rw2, rb2.reshape(1, 16))


def _signnet(pe, edge_index, phi_params, rho_params):
    src = edge_index[0].astype(jnp.int32)
    dst = edge_index[1].astype(jnp.int32)
    pad = EP - E
    pad_dst = N + (jnp.arange(pad, dtype=jnp.int32) % CH)
    src_p = jnp.concatenate([src, jnp.zeros((pad,), jnp.int32)])
    dst_2d = jnp.concatenate([dst, pad_dst]).reshape(EP // CH, CH)
    src_2d = src_p.reshape(EP // CH, CH)
    srcq_2d = jnp.stack([src_2d, src_2d + N])
    dstb_2d = jnp.broadcast_to(dst_2d, (2, EP // CH, CH))
    idx64 = jnp.stack([srcq_2d, dstb_2d], axis=2).reshape(2 * EP // CH, 2, CH)
    zeros32 = jnp.zeros((NA, 32), jnp.float32)
    zeros16 = jnp.zeros((NA, 16), jnp.float32)

    (w11, b11, w12, b12), l2, l3 = phi_params
    eye8 = jnp.eye(8, dtype=jnp.float32)
    bm64 = jnp.kron(jnp.ones((8, 8), jnp.float32), eye8)
    bm32 = jnp.kron(jnp.ones((4, 4), jnp.float32), eye8)

    ztab = jnp.concatenate(
        [jnp.where(jnp.isnan(pe), 0.0, pe), jnp.zeros((N, K), jnp.float32)], axis=1)
    agg1 = _seg8(ztab, idx64, zeros16)
    tp, tm = _dense1(pe, agg1, w11, b11.reshape(1, 8), w12,
                     b12.reshape(1, 8), bm64)

    for li, (w1, b1, w2, b2) in enumerate((l2, l3)):
        outw = w2.shape[1] * 4
        wk1 = w1
        b1t = b1.reshape(1, 8)
        wk2 = w2
        b2t = b2.reshape(1, outw // 4)
        aggp = _seg64(tp.reshape(2 * N, 32), idx64, zeros32)
        aggm = _seg64(tm.reshape(2 * N, 32), idx64, zeros32)
        tp, tm = _dense23(outw, tp, aggp, tm, aggm, wk1, b1t, wk2, b2t, bm32)

    rw1, rb1, rw2, rb2 = rho_params
    return _rho(tp, tm, rw1, rb1, rw2, rb2)


def kernel(x, edge_index, laplacian_pe, batch, edge_attr, Wh, bh, We, be, phi_params, rho_params):
    h = _embed_h(x.astype(jnp.float32), Wh, bh)
    e = _embed_e(edge_attr.astype(jnp.float32), We, be)
    pos_enc = _signnet(laplacian_pe, edge_index, phi_params, rho_params)
    x_new = jnp.concatenate([h, pos_enc], axis=1)
    return x_new, e, pos_enc
